# Initial kernel scaffold; baseline (speedup 1.0000x reference)
import jax
import jax.numpy as jnp
from jax.experimental import pallas as pl


def kernel(x_tweet, x_cls, edge_index_tweet_to_tweet, edge_index_tweet_agg_cls, params):
    raise NotImplementedError("write your pallas kernel here")


# speedup vs baseline: 3.2175x; 3.2175x over previous
import functools

import jax
import jax.numpy as jnp
from jax import lax
from jax.experimental import pallas as pl
from jax.experimental.pallas import tpu as pltpu
from jax.experimental.pallas import tpu_sc as plsc

N_T = 10000
N_C = 5000
D = 256
E_TT = 160000
E_TC = 80000

CH = 128
NCH_TT = E_TT // CH
NCH_TC = E_TC // CH
NSC = 2
NTILE = 16
T_HALF = N_T // NSC
C_HALF = N_C // NSC
T_SH = 5120
C_SH = 2560
ROWS_T = T_SH // NTILE
ROWS_C = C_SH // NTILE

_SC_PARAMS = pltpu.CompilerParams(use_tc_tiling_on_sc=False)


def _tc_pre_t_body(x_ref, wg_ref, as_ref, wr_ref, b_ref,
                   hs_ref, als_ref, xwr_ref):
    x = x_ref[...]
    hs = jnp.dot(x, wg_ref[...], preferred_element_type=jnp.float32)
    hs_ref[...] = hs
    al = (hs * as_ref[...]).sum(-1)
    als_ref[...] = jnp.broadcast_to(al[:, None], (al.shape[0], 16))
    xwr_ref[...] = (jnp.dot(x, wr_ref[...], preferred_element_type=jnp.float32)
                    + b_ref[...])


def _tc_pre_t(x, wg, a_s, wr, b):
    nb = 10
    br = N_T // nb
    return pl.pallas_call(
        _tc_pre_t_body,
        grid=(nb,),
        in_specs=[
            pl.BlockSpec((br, D), lambda i: (i, 0)),
            pl.BlockSpec((D, D), lambda i: (0, 0)),
            pl.BlockSpec((1, D), lambda i: (0, 0)),
            pl.BlockSpec((D, D), lambda i: (0, 0)),
            pl.BlockSpec((1, D), lambda i: (0, 0)),
        ],
        out_specs=[
            pl.BlockSpec((br, D), lambda i: (i, 0)),
            pl.BlockSpec((br, 16), lambda i: (i, 0)),
            pl.BlockSpec((br, D), lambda i: (i, 0)),
        ],
        out_shape=[
            jax.ShapeDtypeStruct((N_T, D), jnp.float32),
            jax.ShapeDtypeStruct((N_T, 16), jnp.float32),
            jax.ShapeDtypeStruct((N_T, D), jnp.float32),
        ],
    )(x, wg, a_s.reshape(1, D), wr, b.reshape(1, D))


def _tc_pre_c_body(x_ref, wg_ref, ad_ref, ald_ref):
    hd = jnp.dot(x_ref[...], wg_ref[...], preferred_element_type=jnp.float32)
    al = (hd * ad_ref[...]).sum(-1)
    ald_ref[...] = jnp.broadcast_to(al[:, None], (al.shape[0], 16))


def _tc_pre_c(xc, wg, a_d):
    nb = 5
    br = N_C // nb
    return pl.pallas_call(
        _tc_pre_c_body,
        grid=(nb,),
        in_specs=[
            pl.BlockSpec((br, D), lambda i: (i, 0)),
            pl.BlockSpec((D, D), lambda i: (0, 0)),
            pl.BlockSpec((1, D), lambda i: (0, 0)),
        ],
        out_specs=pl.BlockSpec((br, 16), lambda i: (i, 0)),
        out_shape=jax.ShapeDtypeStruct((N_C, 16), jnp.float32),
    )(xc, wg, a_d.reshape(1, D))


def _tc_post_t_body(agg_ref, cnt_ref, xwr_ref, wl_ref, out_ref, *, act):
    br = agg_ref.shape[1]
    cnt = jnp.clip(cnt_ref[0, :, 0], 1.0, None)
    mean = agg_ref[0] / cnt[:, None]
    o = jnp.dot(mean, wl_ref[...], preferred_element_type=jnp.float32) \
        + xwr_ref[...]
    if act:
        o = jnp.where(o > 0, o, (jnp.exp(o) - 1.0))
    out_ref[...] = o


def _tc_post_t(agg, cnt_part, xwr, wl, act):
    nb = 10
    br = N_T // nb
    return pl.pallas_call(
        functools.partial(_tc_post_t_body, act=act),
        grid=(nb,),
        in_specs=[
            pl.BlockSpec((1, br, D), lambda i: (i // 5, i % 5, 0)),
            pl.BlockSpec((1, br, 16), lambda i: (i // 5, i % 5, 0)),
            pl.BlockSpec((br, D), lambda i: (i, 0)),
            pl.BlockSpec((D, D), lambda i: (0, 0)),
        ],
        out_specs=pl.BlockSpec((br, D), lambda i: (i, 0)),
        out_shape=jax.ShapeDtypeStruct((N_T, D), jnp.float32),
    )(agg, cnt_part, xwr, wl)


def _tc_post_c_body(acc_ref, den_ref, b_ref, out_ref, *, act):
    den0 = den_ref[0, :C_HALF, 0]
    den1 = den_ref[1, :C_HALF, 0]
    den = jnp.concatenate([den0, den1]) + 1e-16
    acc = jnp.concatenate([acc_ref[0, :C_HALF], acc_ref[1, :C_HALF]])
    o = acc / den[:, None] + b_ref[...]
    if act:
        o = jnp.where(o > 0, o, (jnp.exp(o) - 1.0))
    out_ref[...] = o


def _tc_post_c(acc, den_part, b, act):
    return pl.pallas_call(
        functools.partial(_tc_post_c_body, act=act),
        grid=(1,),
        in_specs=[
            pl.BlockSpec((NSC, C_SH, D), lambda i: (0, 0, 0)),
            pl.BlockSpec((NSC, C_SH, 16), lambda i: (0, 0, 0)),
            pl.BlockSpec((1, D), lambda i: (0, 0)),
        ],
        out_specs=pl.BlockSpec((N_C, D), lambda i: (0, 0)),
        out_shape=jax.ShapeDtypeStruct((N_C, D), jnp.float32),
    )(acc, den_part, b.reshape(1, D))


def _fill_rows(ref, nrows, ncols, val):
    v = jnp.full((16,), val, jnp.float32)

    def z(j, _):
        for m in range(ncols // 16):
            ref[j, pl.ds(16 * m, 16)] = v
        return 0
    lax.fori_loop(0, nrows, z, 0)


def _init_acc(buf, buf_rows, acc_sh, per_tile, tid):
    start = tid * per_tile
    off = 0
    while off < per_tile:
        sz = min(buf_rows, per_tile - off)
        pltpu.sync_copy(buf.at[pl.ds(0, sz)],
                        acc_sh.at[pl.ds(start + off, sz)])
        off += sz


def _out_acc(buf, buf_rows, acc_sh, hbm, per_tile, tid, cid):
    start = tid * per_tile
    off = 0
    while off < per_tile:
        sz = min(buf_rows, per_tile - off)
        pltpu.sync_copy(acc_sh.at[pl.ds(start + off, sz)],
                        buf.at[pl.ds(0, sz)])
        pltpu.sync_copy(buf.at[pl.ds(0, sz)],
                        hbm.at[cid, pl.ds(start + off, sz), :])
        off += sz


def _sc_sage_body(xt, ei_tt, agg_out, cnt_out,
                  idx_a, idx_b, idx_c, rows_v, ones_v,
                  agg_sh, cnt_sh):
    cid = lax.axis_index("c")
    tid = lax.axis_index("s")
    t_base = cid * T_HALF

    _fill_rows(rows_v, CH, D, 0.0)
    _fill_rows(ones_v, CH, 16, 0.0)
    _init_acc(rows_v, CH, agg_sh, ROWS_T, tid)
    _init_acc(ones_v, CH, cnt_sh, ROWS_T, tid)
    _fill_rows(ones_v, CH, 16, 1.0)
    plsc.subcore_barrier()

    n_my = (NCH_TT - tid + NTILE - 1) // NTILE

    def sage_chunk(i, _):
        base = (tid + i * NTILE) * CH
        pltpu.sync_copy(ei_tt.at[0, pl.ds(base, CH)], idx_a)
        pltpu.sync_copy(ei_tt.at[1, pl.ds(base, CH)], idx_b)
        for k in range(CH // 16):
            d16 = idx_b[pl.ds(16 * k, 16)]
            ld = d16 - t_base
            ok = (ld >= 0) & (ld < T_HALF)
            idx_c[pl.ds(16 * k, 16)] = jnp.where(ok, ld, T_HALF)
        pltpu.sync_copy(xt.at[idx_a], rows_v)
        pltpu.sync_copy(rows_v, agg_sh.at[idx_c], add=True)
        pltpu.sync_copy(ones_v, cnt_sh.at[idx_c], add=True)
        return 0

    lax.fori_loop(0, n_my, sage_chunk, 0)
    plsc.subcore_barrier()

    _out_acc(rows_v, CH, agg_sh, agg_out, ROWS_T, tid, cid)
    _out_acc(ones_v, CH, cnt_sh, cnt_out, ROWS_T, tid, cid)


def _sc_sage(xt, ei_tt):
    mesh = plsc.VectorSubcoreMesh(core_axis_name="c", subcore_axis_name="s")
    fn = pl.kernel(
        _sc_sage_body,
        out_type=[
            jax.ShapeDtypeStruct((NSC, T_SH, D), jnp.float32),
            jax.ShapeDtypeStruct((NSC, T_SH, 16), jnp.float32),
        ],
        mesh=mesh,
        compiler_params=_SC_PARAMS,
        scratch_types=[
            pltpu.VMEM((CH,), jnp.int32),
            pltpu.VMEM((CH,), jnp.int32),
            pltpu.VMEM((CH,), jnp.int32),
            pltpu.VMEM((CH, D), jnp.float32),
            pltpu.VMEM((CH, 16), jnp.float32),
            pltpu.VMEM_SHARED((T_SH, D), jnp.float32),
            pltpu.VMEM_SHARED((T_SH, 16), jnp.float32),
        ],
    )
    return fn(xt, ei_tt)


def _sc_gat_body(hs, ei_tc, als_hbm, ald_hbm, gat_out, den_out,
                 idx_a, idx_b, idx_c, rows_v, erep_s, erep_d, mbuf,
                 gat_sh, den_sh, max_xc):
    cid = lax.axis_index("c")
    tid = lax.axis_index("s")
    c_base = cid * C_HALF

    _fill_rows(rows_v, CH, D, 0.0)
    _fill_rows(erep_s, CH, 16, 0.0)
    _init_acc(rows_v, CH, gat_sh, ROWS_C, tid)
    _init_acc(erep_s, CH, den_sh, ROWS_C, tid)
    plsc.subcore_barrier()

    n_my_g = (NCH_TC - tid + NTILE - 1) // NTILE

    def gat1_chunk(i, mx):
        base = (tid + i * NTILE) * CH
        pltpu.sync_copy(ei_tc.at[0, pl.ds(base, CH)], idx_a)
        pltpu.sync_copy(ei_tc.at[1, pl.ds(base, CH)], idx_b)
        pltpu.sync_copy(als_hbm.at[idx_a], erep_s)
        pltpu.sync_copy(ald_hbm.at[idx_b], erep_d)

        def row_max(r, mxi):
            e = erep_s[r, pl.ds(0, 16)] + erep_d[r, pl.ds(0, 16)]
            e = jnp.maximum(e, 0.2 * e)
            return jnp.maximum(mxi, e)
        return lax.fori_loop(0, CH, row_max, mx)

    mx = lax.fori_loop(0, n_my_g, gat1_chunk,
                       jnp.full((16,), -3e38, jnp.float32))

    mbuf[0] = mx
    pltpu.sync_copy(mbuf.at[0], max_xc.at[tid])
    plsc.subcore_barrier()
    pltpu.sync_copy(max_xc, mbuf)
    for r in range(NTILE):
        mx = jnp.maximum(mx, mbuf[r])
    m_sc = mx

    def gat2_chunk(i, _):
        base = (tid + i * NTILE) * CH
        pltpu.sync_copy(ei_tc.at[0, pl.ds(base, CH)], idx_a)
        pltpu.sync_copy(ei_tc.at[1, pl.ds(base, CH)], idx_b)
        pltpu.sync_copy(als_hbm.at[idx_a], erep_s)
        pltpu.sync_copy(ald_hbm.at[idx_b], erep_d)
        for k in range(CH // 16):
            d16 = idx_b[pl.ds(16 * k, 16)]
            ld = d16 - c_base
            ok = (ld >= 0) & (ld < C_HALF)
            idx_c[pl.ds(16 * k, 16)] = jnp.where(ok, ld, C_HALF)
        pltpu.sync_copy(hs.at[idx_a], rows_v)

        def row_scale(r, _):
            e = erep_s[r, pl.ds(0, 16)] + erep_d[r, pl.ds(0, 16)]
            e = jnp.maximum(e, 0.2 * e)
            ex = jnp.exp(e - m_sc)
            erep_s[r, pl.ds(0, 16)] = ex
            for m in range(D // 16):
                rows_v[r, pl.ds(16 * m, 16)] = ex * rows_v[r, pl.ds(16 * m, 16)]
            return 0
        lax.fori_loop(0, CH, row_scale, 0)
        pltpu.sync_copy(rows_v, gat_sh.at[idx_c], add=True)
        pltpu.sync_copy(erep_s, den_sh.at[idx_c], add=True)
        return 0

    lax.fori_loop(0, n_my_g, gat2_chunk, 0)
    plsc.subcore_barrier()

    _out_acc(rows_v, CH, gat_sh, gat_out, ROWS_C, tid, cid)
    _out_acc(erep_s, CH, den_sh, den_out, ROWS_C, tid, cid)


def _sc_gat(hs, ei_tc, als_rep, ald_rep):
    mesh = plsc.VectorSubcoreMesh(core_axis_name="c", subcore_axis_name="s")
    fn = pl.kernel(
        _sc_gat_body,
        out_type=[
            jax.ShapeDtypeStruct((NSC, C_SH, D), jnp.float32),
            jax.ShapeDtypeStruct((NSC, C_SH, 16), jnp.float32),
        ],
        mesh=mesh,
        compiler_params=_SC_PARAMS,
        scratch_types=[
            pltpu.VMEM((CH,), jnp.int32),
            pltpu.VMEM((CH,), jnp.int32),
            pltpu.VMEM((CH,), jnp.int32),
            pltpu.VMEM((CH, D), jnp.float32),
            pltpu.VMEM((CH, 16), jnp.float32),
            pltpu.VMEM((CH, 16), jnp.float32),
            pltpu.VMEM((NTILE, 16), jnp.float32),
            pltpu.VMEM_SHARED((C_SH, D), jnp.float32),
            pltpu.VMEM_SHARED((C_SH, 16), jnp.float32),
            pltpu.VMEM_SHARED((NTILE, 16), jnp.float32),
        ],
    )
    return fn(hs, ei_tc, als_rep, ald_rep)


def kernel(x_tweet, x_cls, edge_index_tweet_to_tweet, edge_index_tweet_agg_cls,
           params):
    ei_tt = edge_index_tweet_to_tweet.astype(jnp.int32)
    ei_tc = edge_index_tweet_agg_cls.astype(jnp.int32)
    xt, xc = x_tweet, x_cls
    n = len(params)
    for i, p in enumerate(params):
        hs, als_rep, xwr = _tc_pre_t(xt, p['gat_W'], p['gat_as'],
                                     p['sage_Wr'], p['sage_b'])
        ald_rep = _tc_pre_c(xc, p['gat_W'], p['gat_ad'])
        agg, cnt_part = _sc_sage(xt, ei_tt)
        gat_acc, den_part = _sc_gat(hs, ei_tc, als_rep, ald_rep)
        act = i < n - 1
        xt = _tc_post_t(agg, cnt_part, xwr, p['sage_Wl'], act)
        xc = _tc_post_c(gat_acc, den_part, p['gat_b'], act)
    return xt, xc


# speedup vs baseline: 3.7824x; 1.1756x over previous
import functools

import jax
import jax.numpy as jnp
from jax import lax
from jax.experimental import pallas as pl
from jax.experimental.pallas import tpu as pltpu
from jax.experimental.pallas import tpu_sc as plsc

N_T = 10000
N_C = 5000
D = 256
E_TT = 160000
E_TC = 80000

CHS = 128
CHG = 160
NCH_TT = E_TT // CHS
NCH_TC = E_TC // CHG
NSC = 2
NTILE = 16
T_HALF = N_T // NSC
C_HALF = N_C // NSC
T_SH = 5120
C_SH = 2560
ROWS_T = T_SH // NTILE
ROWS_C = C_SH // NTILE

_SC_PARAMS = pltpu.CompilerParams(use_tc_tiling_on_sc=False)


def _tc_pre_t_body(x_ref, wg_ref, as_ref, wr_ref, b_ref,
                   hs_ref, als_ref, amax_ref, xwr_ref):
    x = x_ref[...]
    hs = jnp.dot(x, wg_ref[...], preferred_element_type=jnp.float32)
    hs_ref[...] = hs
    al = (hs * as_ref[...]).sum(-1)
    als_ref[...] = jnp.broadcast_to(al[:, None], (al.shape[0], 16))
    i = pl.program_id(0)
    amax_ref[pl.ds(i, 1), :] = jnp.broadcast_to(al.max(), (1, 16))
    xwr_ref[...] = (jnp.dot(x, wr_ref[...], preferred_element_type=jnp.float32)
                    + b_ref[...])


def _tc_pre_t(x, wg, a_s, wr, b):
    nb = 10
    br = N_T // nb
    return pl.pallas_call(
        _tc_pre_t_body,
        grid=(nb,),
        in_specs=[
            pl.BlockSpec((br, D), lambda i: (i, 0)),
            pl.BlockSpec((D, D), lambda i: (0, 0)),
            pl.BlockSpec((1, D), lambda i: (0, 0)),
            pl.BlockSpec((D, D), lambda i: (0, 0)),
            pl.BlockSpec((1, D), lambda i: (0, 0)),
        ],
        out_specs=[
            pl.BlockSpec((br, D), lambda i: (i, 0)),
            pl.BlockSpec((br, 16), lambda i: (i, 0)),
            pl.BlockSpec((nb, 16), lambda i: (0, 0)),
            pl.BlockSpec((br, D), lambda i: (i, 0)),
        ],
        out_shape=[
            jax.ShapeDtypeStruct((N_T, D), jnp.float32),
            jax.ShapeDtypeStruct((N_T, 16), jnp.float32),
            jax.ShapeDtypeStruct((nb, 16), jnp.float32),
            jax.ShapeDtypeStruct((N_T, D), jnp.float32),
        ],
    )(x, wg, a_s.reshape(1, D), wr, b.reshape(1, D))


def _tc_pre_c_body(x_ref, wg_ref, ad_ref, ald_ref, amax_ref):
    hd = jnp.dot(x_ref[...], wg_ref[...], preferred_element_type=jnp.float32)
    al = (hd * ad_ref[...]).sum(-1)
    ald_ref[...] = jnp.broadcast_to(al[:, None], (al.shape[0], 16))
    i = pl.program_id(0)
    amax_ref[pl.ds(i, 1), :] = jnp.broadcast_to(al.max(), (1, 16))


def _tc_pre_c(xc, wg, a_d):
    nb = 5
    br = N_C // nb
    return pl.pallas_call(
        _tc_pre_c_body,
        grid=(nb,),
        in_specs=[
            pl.BlockSpec((br, D), lambda i: (i, 0)),
            pl.BlockSpec((D, D), lambda i: (0, 0)),
            pl.BlockSpec((1, D), lambda i: (0, 0)),
        ],
        out_specs=[
            pl.BlockSpec((br, 16), lambda i: (i, 0)),
            pl.BlockSpec((nb, 16), lambda i: (0, 0)),
        ],
        out_shape=[
            jax.ShapeDtypeStruct((N_C, 16), jnp.float32),
            jax.ShapeDtypeStruct((nb, 16), jnp.float32),
        ],
    )(xc, wg, a_d.reshape(1, D))


def _tc_post_t_body(agg_ref, cnt_ref, xwr_ref, wl_ref, out_ref, *, act):
    br = agg_ref.shape[1]
    cnt = jnp.clip(cnt_ref[0, :, 0], 1.0, None)
    mean = agg_ref[0] / cnt[:, None]
    o = jnp.dot(mean, wl_ref[...], preferred_element_type=jnp.float32) \
        + xwr_ref[...]
    if act:
        o = jnp.where(o > 0, o, (jnp.exp(o) - 1.0))
    out_ref[...] = o


def _tc_post_t(agg, cnt_part, xwr, wl, act):
    nb = 10
    br = N_T // nb
    return pl.pallas_call(
        functools.partial(_tc_post_t_body, act=act),
        grid=(nb,),
        in_specs=[
            pl.BlockSpec((1, br, D), lambda i: (i // 5, i % 5, 0)),
            pl.BlockSpec((1, br, 16), lambda i: (i // 5, i % 5, 0)),
            pl.BlockSpec((br, D), lambda i: (i, 0)),
            pl.BlockSpec((D, D), lambda i: (0, 0)),
        ],
        out_specs=pl.BlockSpec((br, D), lambda i: (i, 0)),
        out_shape=jax.ShapeDtypeStruct((N_T, D), jnp.float32),
    )(agg, cnt_part, xwr, wl)


def _tc_post_c_body(acc_ref, den_ref, b_ref, out_ref, *, act):
    den0 = den_ref[0, :C_HALF, 0]
    den1 = den_ref[1, :C_HALF, 0]
    den = jnp.concatenate([den0, den1]) + 1e-16
    acc = jnp.concatenate([acc_ref[0, :C_HALF], acc_ref[1, :C_HALF]])
    o = acc / den[:, None] + b_ref[...]
    if act:
        o = jnp.where(o > 0, o, (jnp.exp(o) - 1.0))
    out_ref[...] = o


def _tc_post_c(acc, den_part, b, act):
    return pl.pallas_call(
        functools.partial(_tc_post_c_body, act=act),
        grid=(1,),
        in_specs=[
            pl.BlockSpec((NSC, C_SH, D), lambda i: (0, 0, 0)),
            pl.BlockSpec((NSC, C_SH, 16), lambda i: (0, 0, 0)),
            pl.BlockSpec((1, D), lambda i: (0, 0)),
        ],
        out_specs=pl.BlockSpec((N_C, D), lambda i: (0, 0)),
        out_shape=jax.ShapeDtypeStruct((N_C, D), jnp.float32),
    )(acc, den_part, b.reshape(1, D))


def _fill_rows(ref, nrows, ncols, val):
    v = jnp.full((16,), val, jnp.float32)

    def z(j, _):
        for m in range(ncols // 16):
            ref[j, pl.ds(16 * m, 16)] = v
        return 0
    lax.fori_loop(0, nrows, z, 0)


def _init_acc(buf, buf_rows, acc_sh, per_tile, tid):
    start = tid * per_tile
    off = 0
    while off < per_tile:
        sz = min(buf_rows, per_tile - off)
        pltpu.sync_copy(buf.at[pl.ds(0, sz)],
                        acc_sh.at[pl.ds(start + off, sz)])
        off += sz


def _out_acc(buf, buf_rows, acc_sh, hbm, per_tile, tid, cid):
    start = tid * per_tile
    off = 0
    while off < per_tile:
        sz = min(buf_rows, per_tile - off)
        pltpu.sync_copy(acc_sh.at[pl.ds(start + off, sz)],
                        buf.at[pl.ds(0, sz)])
        pltpu.sync_copy(buf.at[pl.ds(0, sz)],
                        hbm.at[cid, pl.ds(start + off, sz), :])
        off += sz


def _sc_sage_body(xt, ei_tt, agg_out, cnt_out,
                  idx_a, idx_b, idx_c, rows_v, ones_v,
                  agg_sh, cnt_sh):
    cid = lax.axis_index("c")
    tid = lax.axis_index("s")
    t_base = cid * T_HALF

    _fill_rows(rows_v, CHS, D, 0.0)
    _fill_rows(ones_v, CHS, 16, 0.0)
    _init_acc(rows_v, CHS, agg_sh, ROWS_T, tid)
    _init_acc(ones_v, CHS, cnt_sh, ROWS_T, tid)
    _fill_rows(ones_v, CHS, 16, 1.0)
    plsc.subcore_barrier()

    n_my = (NCH_TT - tid + NTILE - 1) // NTILE

    def sage_chunk(i, _):
        base = (tid + i * NTILE) * CHS
        pltpu.sync_copy(ei_tt.at[0, pl.ds(base, CHS)], idx_a)
        pltpu.sync_copy(ei_tt.at[1, pl.ds(base, CHS)], idx_b)
        for k in range(CHS // 16):
            d16 = idx_b[pl.ds(16 * k, 16)]
            ld = d16 - t_base
            ok = (ld >= 0) & (ld < T_HALF)
            idx_c[pl.ds(16 * k, 16)] = jnp.where(ok, ld, T_HALF)
        pltpu.sync_copy(xt.at[idx_a], rows_v)
        pltpu.sync_copy(rows_v, agg_sh.at[idx_c], add=True)
        pltpu.sync_copy(ones_v, cnt_sh.at[idx_c], add=True)
        return 0

    lax.fori_loop(0, n_my, sage_chunk, 0)
    plsc.subcore_barrier()

    _out_acc(rows_v, CHS, agg_sh, agg_out, ROWS_T, tid, cid)
    _out_acc(ones_v, CHS, cnt_sh, cnt_out, ROWS_T, tid, cid)


def _sc_sage(xt, ei_tt):
    mesh = plsc.VectorSubcoreMesh(core_axis_name="c", subcore_axis_name="s")
    fn = pl.kernel(
        _sc_sage_body,
        out_type=[
            jax.ShapeDtypeStruct((NSC, T_SH, D), jnp.float32),
            jax.ShapeDtypeStruct((NSC, T_SH, 16), jnp.float32),
        ],
        mesh=mesh,
        compiler_params=_SC_PARAMS,
        scratch_types=[
            pltpu.VMEM((CHS,), jnp.int32),
            pltpu.VMEM((CHS,), jnp.int32),
            pltpu.VMEM((CHS,), jnp.int32),
            pltpu.VMEM((CHS, D), jnp.float32),
            pltpu.VMEM((CHS, 16), jnp.float32),
            pltpu.VMEM_SHARED((T_SH, D), jnp.float32),
            pltpu.VMEM_SHARED((T_SH, 16), jnp.float32),
        ],
    )
    return fn(xt, ei_tt)


def _sc_gat_body(hs, ei_tc, als_hbm, ald_hbm, amax_t, amax_c,
                 gat_out, den_out,
                 idx_a, idx_b, idx_c, rows_v, erep_s, erep_d, mbuf,
                 gat_sh, den_sh):
    cid = lax.axis_index("c")
    tid = lax.axis_index("s")
    c_base = cid * C_HALF

    _fill_rows(rows_v, CHG, D, 0.0)
    _fill_rows(erep_s, CHG, 16, 0.0)
    _init_acc(rows_v, CHG, gat_sh, ROWS_C, tid)
    _init_acc(erep_s, CHG, den_sh, ROWS_C, tid)

    pltpu.sync_copy(amax_t, mbuf.at[pl.ds(0, 10)])
    pltpu.sync_copy(amax_c, mbuf.at[pl.ds(10, 5)])
    mt = mbuf[0, pl.ds(0, 16)]
    for r in range(1, 10):
        mt = jnp.maximum(mt, mbuf[r, pl.ds(0, 16)])
    mc = mbuf[10, pl.ds(0, 16)]
    for r in range(11, 15):
        mc = jnp.maximum(mc, mbuf[r, pl.ds(0, 16)])
    sm = mt + mc
    m_sc = jnp.maximum(sm, 0.2 * sm)
    plsc.subcore_barrier()

    n_my_g = (NCH_TC - tid + NTILE - 1) // NTILE

    def gat2_chunk(i, _):
        base = (tid + i * NTILE) * CHG
        pltpu.sync_copy(ei_tc.at[0, pl.ds(base, CHG)], idx_a)
        pltpu.sync_copy(ei_tc.at[1, pl.ds(base, CHG)], idx_b)
        pltpu.sync_copy(als_hbm.at[idx_a], erep_s)
        pltpu.sync_copy(ald_hbm.at[idx_b], erep_d)
        for k in range(CHG // 16):
            d16 = idx_b[pl.ds(16 * k, 16)]
            ld = d16 - c_base
            ok = (ld >= 0) & (ld < C_HALF)
            idx_c[pl.ds(16 * k, 16)] = jnp.where(ok, ld, C_HALF)
        pltpu.sync_copy(hs.at[idx_a], rows_v)

        def row_scale(r, _):
            e = erep_s[r, pl.ds(0, 16)] + erep_d[r, pl.ds(0, 16)]
            e = jnp.maximum(e, 0.2 * e)
            ex = jnp.exp(e - m_sc)
            erep_s[r, pl.ds(0, 16)] = ex
            for m in range(D // 16):
                rows_v[r, pl.ds(16 * m, 16)] = ex * rows_v[r, pl.ds(16 * m, 16)]
            return 0
        lax.fori_loop(0, CHG, row_scale, 0)
        pltpu.sync_copy(rows_v, gat_sh.at[idx_c], add=True)
        pltpu.sync_copy(erep_s, den_sh.at[idx_c], add=True)
        return 0

    lax.fori_loop(0, n_my_g, gat2_chunk, 0)
    plsc.subcore_barrier()

    _out_acc(rows_v, CHG, gat_sh, gat_out, ROWS_C, tid, cid)
    _out_acc(erep_s, CHG, den_sh, den_out, ROWS_C, tid, cid)


def _sc_gat(hs, ei_tc, als_rep, ald_rep, amax_t, amax_c):
    mesh = plsc.VectorSubcoreMesh(core_axis_name="c", subcore_axis_name="s")
    fn = pl.kernel(
        _sc_gat_body,
        out_type=[
            jax.ShapeDtypeStruct((NSC, C_SH, D), jnp.float32),
            jax.ShapeDtypeStruct((NSC, C_SH, 16), jnp.float32),
        ],
        mesh=mesh,
        compiler_params=_SC_PARAMS,
        scratch_types=[
            pltpu.VMEM((CHG,), jnp.int32),
            pltpu.VMEM((CHG,), jnp.int32),
            pltpu.VMEM((CHG,), jnp.int32),
            pltpu.VMEM((CHG, D), jnp.float32),
            pltpu.VMEM((CHG, 16), jnp.float32),
            pltpu.VMEM((CHG, 16), jnp.float32),
            pltpu.VMEM((NTILE, 16), jnp.float32),
            pltpu.VMEM_SHARED((C_SH, D), jnp.float32),
            pltpu.VMEM_SHARED((C_SH, 16), jnp.float32),
        ],
    )
    return fn(hs, ei_tc, als_rep, ald_rep, amax_t, amax_c)


def kernel(x_tweet, x_cls, edge_index_tweet_to_tweet, edge_index_tweet_agg_cls,
           params):
    ei_tt = edge_index_tweet_to_tweet.astype(jnp.int32)
    ei_tc = edge_index_tweet_agg_cls.astype(jnp.int32)
    xt, xc = x_tweet, x_cls
    n = len(params)
    for i, p in enumerate(params):
        hs, als_rep, amax_t, xwr = _tc_pre_t(xt, p['gat_W'], p['gat_as'],
                                             p['sage_Wr'], p['sage_b'])
        ald_rep, amax_c = _tc_pre_c(xc, p['gat_W'], p['gat_ad'])
        agg, cnt_part = _sc_sage(xt, ei_tt)
        gat_acc, den_part = _sc_gat(hs, ei_tc, als_rep, ald_rep,
                                    amax_t, amax_c)
        act = i < n - 1
        xt = _tc_post_t(agg, cnt_part, xwr, p['sage_Wl'], act)
        xc = _tc_post_c(gat_acc, den_part, p['gat_b'], act)
    return xt, xc


# speedup vs baseline: 4.2732x; 1.1297x over previous
import functools

import jax
import jax.numpy as jnp
from jax import lax
from jax.experimental import pallas as pl
from jax.experimental.pallas import tpu as pltpu
from jax.experimental.pallas import tpu_sc as plsc

N_T = 10000
N_C = 5000
D = 256
E_TT = 160000
E_TC = 80000

CHS = 128
CHG = 160
NCH_TT = E_TT // CHS
NCH_TC = E_TC // CHG
NSC = 2
NTILE = 16
T_HALF = N_T // NSC
C_HALF = N_C // NSC
T_SH = 5120
C_SH = 2560
ROWS_T = T_SH // NTILE
ROWS_C = C_SH // NTILE

_SC_PARAMS = pltpu.CompilerParams(use_tc_tiling_on_sc=False)


def _tc_pre_t_body(x_ref, wg_ref, as_ref, wr_ref, b_ref,
                   hs_ref, als_ref, amax_ref, xwr_ref):
    x = x_ref[...]
    hs = jnp.dot(x, wg_ref[...], preferred_element_type=jnp.float32)
    hs_ref[...] = hs
    al = (hs * as_ref[...]).sum(-1)
    als_ref[...] = jnp.broadcast_to(al[:, None], (al.shape[0], 16))
    i = pl.program_id(0)
    amax_ref[pl.ds(i, 1), :] = jnp.broadcast_to(al.max(), (1, 16))
    xwr_ref[...] = (jnp.dot(x, wr_ref[...], preferred_element_type=jnp.float32)
                    + b_ref[...])


def _tc_pre_t(x, wg, a_s, wr, b):
    nb = 10
    br = N_T // nb
    return pl.pallas_call(
        _tc_pre_t_body,
        grid=(nb,),
        in_specs=[
            pl.BlockSpec((br, D), lambda i: (i, 0)),
            pl.BlockSpec((D, D), lambda i: (0, 0)),
            pl.BlockSpec((1, D), lambda i: (0, 0)),
            pl.BlockSpec((D, D), lambda i: (0, 0)),
            pl.BlockSpec((1, D), lambda i: (0, 0)),
        ],
        out_specs=[
            pl.BlockSpec((br, D), lambda i: (i, 0)),
            pl.BlockSpec((br, 16), lambda i: (i, 0)),
            pl.BlockSpec((nb, 16), lambda i: (0, 0)),
            pl.BlockSpec((br, D), lambda i: (i, 0)),
        ],
        out_shape=[
            jax.ShapeDtypeStruct((N_T, D), jnp.float32),
            jax.ShapeDtypeStruct((N_T, 16), jnp.float32),
            jax.ShapeDtypeStruct((nb, 16), jnp.float32),
            jax.ShapeDtypeStruct((N_T, D), jnp.float32),
        ],
    )(x, wg, a_s.reshape(1, D), wr, b.reshape(1, D))


def _tc_pre_c_body(x_ref, wg_ref, ad_ref, ald_ref, amax_ref):
    hd = jnp.dot(x_ref[...], wg_ref[...], preferred_element_type=jnp.float32)
    al = (hd * ad_ref[...]).sum(-1)
    ald_ref[...] = jnp.broadcast_to(al[:, None], (al.shape[0], 16))
    i = pl.program_id(0)
    amax_ref[pl.ds(i, 1), :] = jnp.broadcast_to(al.max(), (1, 16))


def _tc_pre_c(xc, wg, a_d):
    nb = 5
    br = N_C // nb
    return pl.pallas_call(
        _tc_pre_c_body,
        grid=(nb,),
        in_specs=[
            pl.BlockSpec((br, D), lambda i: (i, 0)),
            pl.BlockSpec((D, D), lambda i: (0, 0)),
            pl.BlockSpec((1, D), lambda i: (0, 0)),
        ],
        out_specs=[
            pl.BlockSpec((br, 16), lambda i: (i, 0)),
            pl.BlockSpec((nb, 16), lambda i: (0, 0)),
        ],
        out_shape=[
            jax.ShapeDtypeStruct((N_C, 16), jnp.float32),
            jax.ShapeDtypeStruct((nb, 16), jnp.float32),
        ],
    )(xc, wg, a_d.reshape(1, D))


def _tc_post_t_body(agg_ref, cnt_ref, xwr_ref, wl_ref, out_ref, *, act):
    br = agg_ref.shape[1]
    cnt = jnp.clip(cnt_ref[0, :, 0], 1.0, None)
    mean = agg_ref[0] / cnt[:, None]
    o = jnp.dot(mean, wl_ref[...], preferred_element_type=jnp.float32) \
        + xwr_ref[...]
    if act:
        o = jnp.where(o > 0, o, (jnp.exp(o) - 1.0))
    out_ref[...] = o


def _tc_post_t(agg, cnt_part, xwr, wl, act):
    nb = 10
    br = N_T // nb
    return pl.pallas_call(
        functools.partial(_tc_post_t_body, act=act),
        grid=(nb,),
        in_specs=[
            pl.BlockSpec((1, br, D), lambda i: (i // 5, i % 5, 0)),
            pl.BlockSpec((1, br, 16), lambda i: (i // 5, i % 5, 0)),
            pl.BlockSpec((br, D), lambda i: (i, 0)),
            pl.BlockSpec((D, D), lambda i: (0, 0)),
        ],
        out_specs=pl.BlockSpec((br, D), lambda i: (i, 0)),
        out_shape=jax.ShapeDtypeStruct((N_T, D), jnp.float32),
    )(agg, cnt_part, xwr, wl)


def _tc_post_c_body(acc_ref, den_ref, b_ref, out_ref, *, act):
    den0 = den_ref[0, :C_HALF, 0]
    den1 = den_ref[1, :C_HALF, 0]
    den = jnp.concatenate([den0, den1]) + 1e-16
    acc = jnp.concatenate([acc_ref[0, :C_HALF], acc_ref[1, :C_HALF]])
    o = acc / den[:, None] + b_ref[...]
    if act:
        o = jnp.where(o > 0, o, (jnp.exp(o) - 1.0))
    out_ref[...] = o


def _tc_post_c(acc, den_part, b, act):
    return pl.pallas_call(
        functools.partial(_tc_post_c_body, act=act),
        grid=(1,),
        in_specs=[
            pl.BlockSpec((NSC, C_SH, D), lambda i: (0, 0, 0)),
            pl.BlockSpec((NSC, C_SH, 16), lambda i: (0, 0, 0)),
            pl.BlockSpec((1, D), lambda i: (0, 0)),
        ],
        out_specs=pl.BlockSpec((N_C, D), lambda i: (0, 0)),
        out_shape=jax.ShapeDtypeStruct((N_C, D), jnp.float32),
    )(acc, den_part, b.reshape(1, D))


def _fill_rows(ref, nrows, ncols, val):
    v = jnp.full((16,), val, jnp.float32)

    def z(j, _):
        for m in range(ncols // 16):
            ref[j, pl.ds(16 * m, 16)] = v
        return 0
    lax.fori_loop(0, nrows, z, 0)


def _init_acc(buf, buf_rows, acc_sh, per_tile, tid):
    start = tid * per_tile
    off = 0
    while off < per_tile:
        sz = min(buf_rows, per_tile - off)
        pltpu.sync_copy(buf.at[pl.ds(0, sz)],
                        acc_sh.at[pl.ds(start + off, sz)])
        off += sz


def _out_acc(buf, buf_rows, acc_sh, hbm, per_tile, tid, cid):
    start = tid * per_tile
    off = 0
    while off < per_tile:
        sz = min(buf_rows, per_tile - off)
        pltpu.sync_copy(acc_sh.at[pl.ds(start + off, sz)],
                        buf.at[pl.ds(0, sz)])
        pltpu.sync_copy(buf.at[pl.ds(0, sz)],
                        hbm.at[cid, pl.ds(start + off, sz), :])
        off += sz


def _sc_sage_body(xt, ei_tt, agg_out, cnt_out,
                  idx_a, idx_b, idx_c, rows_v, ones_v,
                  s1, s2, s3, agg_sh, cnt_sh):
    cid = lax.axis_index("c")
    tid = lax.axis_index("s")
    t_base = cid * T_HALF

    _fill_rows(rows_v, CHS, D, 0.0)
    _fill_rows(ones_v, CHS, 16, 0.0)
    _init_acc(rows_v, CHS, agg_sh, ROWS_T, tid)
    _init_acc(ones_v, CHS, cnt_sh, ROWS_T, tid)
    _fill_rows(ones_v, CHS, 16, 1.0)
    plsc.subcore_barrier()

    n_my = (NCH_TT - tid + NTILE - 1) // NTILE

    def sage_chunk(i, _):
        base = (tid + i * NTILE) * CHS
        c1 = pltpu.async_copy(ei_tt.at[0, pl.ds(base, CHS)], idx_a, s1)
        c2 = pltpu.async_copy(ei_tt.at[1, pl.ds(base, CHS)], idx_b, s2)
        c1.wait()
        c3 = pltpu.async_copy(xt.at[idx_a], rows_v, s3)
        c2.wait()
        for k in range(CHS // 16):
            d16 = idx_b[pl.ds(16 * k, 16)]
            ld = d16 - t_base
            ok = (ld >= 0) & (ld < T_HALF)
            idx_c[pl.ds(16 * k, 16)] = jnp.where(ok, ld, T_HALF)
        c3.wait()
        c4 = pltpu.async_copy(rows_v, agg_sh.at[idx_c], s1, add=True)
        c5 = pltpu.async_copy(ones_v, cnt_sh.at[idx_c], s2, add=True)
        c4.wait()
        c5.wait()
        return 0

    lax.fori_loop(0, n_my, sage_chunk, 0)
    plsc.subcore_barrier()

    _out_acc(rows_v, CHS, agg_sh, agg_out, ROWS_T, tid, cid)
    _out_acc(ones_v, CHS, cnt_sh, cnt_out, ROWS_T, tid, cid)


def _sc_sage(xt, ei_tt):
    mesh = plsc.VectorSubcoreMesh(core_axis_name="c", subcore_axis_name="s")
    fn = pl.kernel(
        _sc_sage_body,
        out_type=[
            jax.ShapeDtypeStruct((NSC, T_SH, D), jnp.float32),
            jax.ShapeDtypeStruct((NSC, T_SH, 16), jnp.float32),
        ],
        mesh=mesh,
        compiler_params=_SC_PARAMS,
        scratch_types=[
            pltpu.VMEM((CHS,), jnp.int32),
            pltpu.VMEM((CHS,), jnp.int32),
            pltpu.VMEM((CHS,), jnp.int32),
            pltpu.VMEM((CHS, D), jnp.float32),
            pltpu.VMEM((CHS, 16), jnp.float32),
            pltpu.SemaphoreType.DMA,
            pltpu.SemaphoreType.DMA,
            pltpu.SemaphoreType.DMA,
            pltpu.VMEM_SHARED((T_SH, D), jnp.float32),
            pltpu.VMEM_SHARED((T_SH, 16), jnp.float32),
        ],
    )
    return fn(xt, ei_tt)


def _sc_gat_body(hs, ei_tc, als_hbm, ald_hbm, amax_t, amax_c,
                 gat_out, den_out,
                 idx_a, idx_b, idx_c, rows_v, erep_s, erep_d, mbuf,
                 s1, s2, s3, s4, s5, gat_sh, den_sh):
    cid = lax.axis_index("c")
    tid = lax.axis_index("s")
    c_base = cid * C_HALF

    _fill_rows(rows_v, CHG, D, 0.0)
    _fill_rows(erep_s, CHG, 16, 0.0)
    _init_acc(rows_v, CHG, gat_sh, ROWS_C, tid)
    _init_acc(erep_s, CHG, den_sh, ROWS_C, tid)

    pltpu.sync_copy(amax_t, mbuf.at[pl.ds(0, 10)])
    pltpu.sync_copy(amax_c, mbuf.at[pl.ds(10, 5)])
    mt = mbuf[0, pl.ds(0, 16)]
    for r in range(1, 10):
        mt = jnp.maximum(mt, mbuf[r, pl.ds(0, 16)])
    mc = mbuf[10, pl.ds(0, 16)]
    for r in range(11, 15):
        mc = jnp.maximum(mc, mbuf[r, pl.ds(0, 16)])
    sm = mt + mc
    m_sc = jnp.maximum(sm, 0.2 * sm)
    plsc.subcore_barrier()

    n_my_g = (NCH_TC - tid + NTILE - 1) // NTILE

    def gat2_chunk(i, _):
        base = (tid + i * NTILE) * CHG
        c1 = pltpu.async_copy(ei_tc.at[0, pl.ds(base, CHG)], idx_a, s1)
        c2 = pltpu.async_copy(ei_tc.at[1, pl.ds(base, CHG)], idx_b, s2)
        c1.wait()
        c3 = pltpu.async_copy(hs.at[idx_a], rows_v, s3)
        c4 = pltpu.async_copy(als_hbm.at[idx_a], erep_s, s4)
        c2.wait()
        c5 = pltpu.async_copy(ald_hbm.at[idx_b], erep_d, s5)
        for k in range(CHG // 16):
            d16 = idx_b[pl.ds(16 * k, 16)]
            ld = d16 - c_base
            ok = (ld >= 0) & (ld < C_HALF)
            idx_c[pl.ds(16 * k, 16)] = jnp.where(ok, ld, C_HALF)
        c3.wait()
        c4.wait()
        c5.wait()

        def row_scale(r, _):
            e = erep_s[r, pl.ds(0, 16)] + erep_d[r, pl.ds(0, 16)]
            e = jnp.maximum(e, 0.2 * e)
            ex = jnp.exp(e - m_sc)
            erep_s[r, pl.ds(0, 16)] = ex
            for m in range(D // 16):
                rows_v[r, pl.ds(16 * m, 16)] = ex * rows_v[r, pl.ds(16 * m, 16)]
            return 0
        lax.fori_loop(0, CHG, row_scale, 0)
        c6 = pltpu.async_copy(rows_v, gat_sh.at[idx_c], s1, add=True)
        c7 = pltpu.async_copy(erep_s, den_sh.at[idx_c], s2, add=True)
        c6.wait()
        c7.wait()
        return 0

    lax.fori_loop(0, n_my_g, gat2_chunk, 0)
    plsc.subcore_barrier()

    _out_acc(rows_v, CHG, gat_sh, gat_out, ROWS_C, tid, cid)
    _out_acc(erep_s, CHG, den_sh, den_out, ROWS_C, tid, cid)


def _sc_gat(hs, ei_tc, als_rep, ald_rep, amax_t, amax_c):
    mesh = plsc.VectorSubcoreMesh(core_axis_name="c", subcore_axis_name="s")
    fn = pl.kernel(
        _sc_gat_body,
        out_type=[
            jax.ShapeDtypeStruct((NSC, C_SH, D), jnp.float32),
            jax.ShapeDtypeStruct((NSC, C_SH, 16), jnp.float32),
        ],
        mesh=mesh,
        compiler_params=_SC_PARAMS,
        scratch_types=[
            pltpu.VMEM((CHG,), jnp.int32),
            pltpu.VMEM((CHG,), jnp.int32),
            pltpu.VMEM((CHG,), jnp.int32),
            pltpu.VMEM((CHG, D), jnp.float32),
            pltpu.VMEM((CHG, 16), jnp.float32),
            pltpu.VMEM((CHG, 16), jnp.float32),
            pltpu.VMEM((NTILE, 16), jnp.float32),
            pltpu.SemaphoreType.DMA,
            pltpu.SemaphoreType.DMA,
            pltpu.SemaphoreType.DMA,
            pltpu.SemaphoreType.DMA,
            pltpu.SemaphoreType.DMA,
            pltpu.VMEM_SHARED((C_SH, D), jnp.float32),
            pltpu.VMEM_SHARED((C_SH, 16), jnp.float32),
        ],
    )
    return fn(hs, ei_tc, als_rep, ald_rep, amax_t, amax_c)


def kernel(x_tweet, x_cls, edge_index_tweet_to_tweet, edge_index_tweet_agg_cls,
           params):
    ei_tt = edge_index_tweet_to_tweet.astype(jnp.int32)
    ei_tc = edge_index_tweet_agg_cls.astype(jnp.int32)
    xt, xc = x_tweet, x_cls
    n = len(params)
    for i, p in enumerate(params):
        hs, als_rep, amax_t, xwr = _tc_pre_t(xt, p['gat_W'], p['gat_as'],
                                             p['sage_Wr'], p['sage_b'])
        ald_rep, amax_c = _tc_pre_c(xc, p['gat_W'], p['gat_ad'])
        agg, cnt_part = _sc_sage(xt, ei_tt)
        gat_acc, den_part = _sc_gat(hs, ei_tc, als_rep, ald_rep,
                                    amax_t, amax_c)
        act = i < n - 1
        xt = _tc_post_t(agg, cnt_part, xwr, p['sage_Wl'], act)
        xc = _tc_post_c(gat_acc, den_part, p['gat_b'], act)
    return xt, xc


# speedup vs baseline: 4.3336x; 1.0141x over previous
import functools

import jax
import jax.numpy as jnp
from jax import lax
from jax.experimental import pallas as pl
from jax.experimental.pallas import tpu as pltpu
from jax.experimental.pallas import tpu_sc as plsc

N_T = 10000
N_C = 5000
D = 256
E_TT = 160000
E_TC = 80000

CHS = 160
CHG = 160
NCH_TT = E_TT // CHS
NCH_TC = E_TC // CHG
NSC = 2
NTILE = 16
T_HALF = N_T // NSC
C_HALF = N_C // NSC
T_SH = 5120
C_SH = 2560
ROWS_T = T_SH // NTILE
ROWS_C = C_SH // NTILE

_SC_PARAMS = pltpu.CompilerParams(use_tc_tiling_on_sc=False)


def _tc_pre_t_body(x_ref, wg_ref, as_ref, wr_ref, b_ref,
                   hs_ref, als_ref, amax_ref, xwr_ref):
    x = x_ref[...]
    hs = jnp.dot(x, wg_ref[...], preferred_element_type=jnp.float32)
    hs_ref[...] = hs
    al = (hs * as_ref[...]).sum(-1)
    als_ref[...] = jnp.broadcast_to(al[:, None], (al.shape[0], 16))
    i = pl.program_id(0)
    amax_ref[pl.ds(i, 1), :] = jnp.broadcast_to(al.max(), (1, 16))
    xwr_ref[...] = (jnp.dot(x, wr_ref[...], preferred_element_type=jnp.float32)
                    + b_ref[...])


def _tc_pre_t(x, wg, a_s, wr, b):
    nb = 10
    br = N_T // nb
    return pl.pallas_call(
        _tc_pre_t_body,
        grid=(nb,),
        in_specs=[
            pl.BlockSpec((br, D), lambda i: (i, 0)),
            pl.BlockSpec((D, D), lambda i: (0, 0)),
            pl.BlockSpec((1, D), lambda i: (0, 0)),
            pl.BlockSpec((D, D), lambda i: (0, 0)),
            pl.BlockSpec((1, D), lambda i: (0, 0)),
        ],
        out_specs=[
            pl.BlockSpec((br, D), lambda i: (i, 0)),
            pl.BlockSpec((br, 16), lambda i: (i, 0)),
            pl.BlockSpec((nb, 16), lambda i: (0, 0)),
            pl.BlockSpec((br, D), lambda i: (i, 0)),
        ],
        out_shape=[
            jax.ShapeDtypeStruct((N_T, D), jnp.float32),
            jax.ShapeDtypeStruct((N_T, 16), jnp.float32),
            jax.ShapeDtypeStruct((nb, 16), jnp.float32),
            jax.ShapeDtypeStruct((N_T, D), jnp.float32),
        ],
    )(x, wg, a_s.reshape(1, D), wr, b.reshape(1, D))


def _tc_pre_c_body(x_ref, wg_ref, ad_ref, ald_ref, amax_ref):
    hd = jnp.dot(x_ref[...], wg_ref[...], preferred_element_type=jnp.float32)
    al = (hd * ad_ref[...]).sum(-1)
    ald_ref[...] = jnp.broadcast_to(al[:, None], (al.shape[0], 16))
    i = pl.program_id(0)
    amax_ref[pl.ds(i, 1), :] = jnp.broadcast_to(al.max(), (1, 16))


def _tc_pre_c(xc, wg, a_d):
    nb = 5
    br = N_C // nb
    return pl.pallas_call(
        _tc_pre_c_body,
        grid=(nb,),
        in_specs=[
            pl.BlockSpec((br, D), lambda i: (i, 0)),
            pl.BlockSpec((D, D), lambda i: (0, 0)),
            pl.BlockSpec((1, D), lambda i: (0, 0)),
        ],
        out_specs=[
            pl.BlockSpec((br, 16), lambda i: (i, 0)),
            pl.BlockSpec((nb, 16), lambda i: (0, 0)),
        ],
        out_shape=[
            jax.ShapeDtypeStruct((N_C, 16), jnp.float32),
            jax.ShapeDtypeStruct((nb, 16), jnp.float32),
        ],
    )(xc, wg, a_d.reshape(1, D))


def _tc_post_t_body(agg_ref, cnt_ref, xwr_ref, wl_ref, out_ref, *, act):
    br = agg_ref.shape[1]
    cnt = jnp.clip(cnt_ref[0, :, 0], 1.0, None)
    mean = agg_ref[0] / cnt[:, None]
    o = jnp.dot(mean, wl_ref[...], preferred_element_type=jnp.float32) \
        + xwr_ref[...]
    if act:
        o = jnp.where(o > 0, o, (jnp.exp(o) - 1.0))
    out_ref[...] = o


def _tc_post_t(agg, cnt_part, xwr, wl, act):
    nb = 10
    br = N_T // nb
    return pl.pallas_call(
        functools.partial(_tc_post_t_body, act=act),
        grid=(nb,),
        in_specs=[
            pl.BlockSpec((1, br, D), lambda i: (i // 5, i % 5, 0)),
            pl.BlockSpec((1, br, 16), lambda i: (i // 5, i % 5, 0)),
            pl.BlockSpec((br, D), lambda i: (i, 0)),
            pl.BlockSpec((D, D), lambda i: (0, 0)),
        ],
        out_specs=pl.BlockSpec((br, D), lambda i: (i, 0)),
        out_shape=jax.ShapeDtypeStruct((N_T, D), jnp.float32),
    )(agg, cnt_part, xwr, wl)


def _tc_post_c_body(acc_ref, den_ref, b_ref, out_ref, *, act):
    den0 = den_ref[0, :C_HALF, 0]
    den1 = den_ref[1, :C_HALF, 0]
    den = jnp.concatenate([den0, den1]) + 1e-16
    acc = jnp.concatenate([acc_ref[0, :C_HALF], acc_ref[1, :C_HALF]])
    o = acc / den[:, None] + b_ref[...]
    if act:
        o = jnp.where(o > 0, o, (jnp.exp(o) - 1.0))
    out_ref[...] = o


def _tc_post_c(acc, den_part, b, act):
    return pl.pallas_call(
        functools.partial(_tc_post_c_body, act=act),
        grid=(1,),
        in_specs=[
            pl.BlockSpec((NSC, C_SH, D), lambda i: (0, 0, 0)),
            pl.BlockSpec((NSC, C_SH, 16), lambda i: (0, 0, 0)),
            pl.BlockSpec((1, D), lambda i: (0, 0)),
        ],
        out_specs=pl.BlockSpec((N_C, D), lambda i: (0, 0)),
        out_shape=jax.ShapeDtypeStruct((N_C, D), jnp.float32),
    )(acc, den_part, b.reshape(1, D))


def _fill_rows(ref, nrows, ncols, val):
    v = jnp.full((16,), val, jnp.float32)

    def z(j, _):
        for m in range(ncols // 16):
            ref[j, pl.ds(16 * m, 16)] = v
        return 0
    lax.fori_loop(0, nrows, z, 0)


def _init_acc(buf, buf_rows, acc_sh, per_tile, tid):
    start = tid * per_tile
    off = 0
    while off < per_tile:
        sz = min(buf_rows, per_tile - off)
        pltpu.sync_copy(buf.at[pl.ds(0, sz)],
                        acc_sh.at[pl.ds(start + off, sz)])
        off += sz


def _out_acc(buf, buf_rows, acc_sh, hbm, per_tile, tid, cid):
    start = tid * per_tile
    off = 0
    while off < per_tile:
        sz = min(buf_rows, per_tile - off)
        pltpu.sync_copy(acc_sh.at[pl.ds(start + off, sz)],
                        buf.at[pl.ds(0, sz)])
        pltpu.sync_copy(buf.at[pl.ds(0, sz)],
                        hbm.at[cid, pl.ds(start + off, sz), :])
        off += sz


def _sc_sage_body(xt, ei_tt, agg_out, cnt_out,
                  idx_a, idx_b, idx_c, rows_v, ones_v,
                  s1, s2, s3, agg_sh, cnt_sh):
    cid = lax.axis_index("c")
    tid = lax.axis_index("s")
    t_base = cid * T_HALF

    _fill_rows(rows_v, CHS, D, 0.0)
    _fill_rows(ones_v, CHS, 16, 0.0)
    _init_acc(rows_v, CHS, agg_sh, ROWS_T, tid)
    _init_acc(ones_v, CHS, cnt_sh, ROWS_T, tid)
    _fill_rows(ones_v, CHS, 16, 1.0)
    plsc.subcore_barrier()

    n_my = (NCH_TT - tid + NTILE - 1) // NTILE

    def sage_chunk(i, _):
        base = (tid + i * NTILE) * CHS
        c1 = pltpu.async_copy(ei_tt.at[0, pl.ds(base, CHS)], idx_a, s1)
        c2 = pltpu.async_copy(ei_tt.at[1, pl.ds(base, CHS)], idx_b, s2)
        c1.wait()
        c3 = pltpu.async_copy(xt.at[idx_a], rows_v, s3)
        c2.wait()
        for k in range(CHS // 16):
            d16 = idx_b[pl.ds(16 * k, 16)]
            ld = d16 - t_base
            ok = (ld >= 0) & (ld < T_HALF)
            idx_c[pl.ds(16 * k, 16)] = jnp.where(ok, ld, T_HALF)
        c3.wait()
        c4 = pltpu.async_copy(rows_v, agg_sh.at[idx_c], s1, add=True)
        c5 = pltpu.async_copy(ones_v, cnt_sh.at[idx_c], s2, add=True)
        c4.wait()
        c5.wait()
        return 0

    lax.fori_loop(0, n_my, sage_chunk, 0)
    plsc.subcore_barrier()

    _out_acc(rows_v, CHS, agg_sh, agg_out, ROWS_T, tid, cid)
    _out_acc(ones_v, CHS, cnt_sh, cnt_out, ROWS_T, tid, cid)


def _sc_sage(xt, ei_tt):
    mesh = plsc.VectorSubcoreMesh(core_axis_name="c", subcore_axis_name="s")
    fn = pl.kernel(
        _sc_sage_body,
        out_type=[
            jax.ShapeDtypeStruct((NSC, T_SH, D), jnp.float32),
            jax.ShapeDtypeStruct((NSC, T_SH, 16), jnp.float32),
        ],
        mesh=mesh,
        compiler_params=_SC_PARAMS,
        scratch_types=[
            pltpu.VMEM((CHS,), jnp.int32),
            pltpu.VMEM((CHS,), jnp.int32),
            pltpu.VMEM((CHS,), jnp.int32),
            pltpu.VMEM((CHS, D), jnp.float32),
            pltpu.VMEM((CHS, 16), jnp.float32),
            pltpu.SemaphoreType.DMA,
            pltpu.SemaphoreType.DMA,
            pltpu.SemaphoreType.DMA,
            pltpu.VMEM_SHARED((T_SH, D), jnp.float32),
            pltpu.VMEM_SHARED((T_SH, 16), jnp.float32),
        ],
    )
    return fn(xt, ei_tt)


def _sc_gat_body(hs, ei_tc, als_hbm, ald_hbm, amax_t, amax_c,
                 gat_out, den_out,
                 idx_a, idx_b, idx_c, rows_v, erep_s, erep_d, mbuf,
                 s1, s2, s3, s4, s5, gat_sh, den_sh):
    cid = lax.axis_index("c")
    tid = lax.axis_index("s")
    c_base = cid * C_HALF

    _fill_rows(rows_v, CHG, D, 0.0)
    _fill_rows(erep_s, CHG, 16, 0.0)
    _init_acc(rows_v, CHG, gat_sh, ROWS_C, tid)
    _init_acc(erep_s, CHG, den_sh, ROWS_C, tid)

    pltpu.sync_copy(amax_t, mbuf.at[pl.ds(0, 10)])
    pltpu.sync_copy(amax_c, mbuf.at[pl.ds(10, 5)])
    mt = mbuf[0, pl.ds(0, 16)]
    for r in range(1, 10):
        mt = jnp.maximum(mt, mbuf[r, pl.ds(0, 16)])
    mc = mbuf[10, pl.ds(0, 16)]
    for r in range(11, 15):
        mc = jnp.maximum(mc, mbuf[r, pl.ds(0, 16)])
    sm = mt + mc
    m_sc = jnp.maximum(sm, 0.2 * sm)
    plsc.subcore_barrier()

    n_my_g = (NCH_TC - tid + NTILE - 1) // NTILE

    def gat2_chunk(i, _):
        base = (tid + i * NTILE) * CHG
        c1 = pltpu.async_copy(ei_tc.at[0, pl.ds(base, CHG)], idx_a, s1)
        c2 = pltpu.async_copy(ei_tc.at[1, pl.ds(base, CHG)], idx_b, s2)
        c1.wait()
        c3 = pltpu.async_copy(hs.at[idx_a], rows_v, s3)
        c4 = pltpu.async_copy(als_hbm.at[idx_a], erep_s, s4)
        c2.wait()
        c5 = pltpu.async_copy(ald_hbm.at[idx_b], erep_d, s5)
        for k in range(CHG // 16):
            d16 = idx_b[pl.ds(16 * k, 16)]
            ld = d16 - c_base
            ok = (ld >= 0) & (ld < C_HALF)
            idx_c[pl.ds(16 * k, 16)] = jnp.where(ok, ld, C_HALF)
        c3.wait()
        c4.wait()
        c5.wait()

        def row_scale(r, _):
            e = erep_s[r, pl.ds(0, 16)] + erep_d[r, pl.ds(0, 16)]
            e = jnp.maximum(e, 0.2 * e)
            ex = jnp.exp(e - m_sc)
            erep_s[r, pl.ds(0, 16)] = ex
            for m in range(D // 16):
                rows_v[r, pl.ds(16 * m, 16)] = ex * rows_v[r, pl.ds(16 * m, 16)]
            return 0
        lax.fori_loop(0, CHG, row_scale, 0)
        c6 = pltpu.async_copy(rows_v, gat_sh.at[idx_c], s1, add=True)
        c7 = pltpu.async_copy(erep_s, den_sh.at[idx_c], s2, add=True)
        c6.wait()
        c7.wait()
        return 0

    lax.fori_loop(0, n_my_g, gat2_chunk, 0)
    plsc.subcore_barrier()

    _out_acc(rows_v, CHG, gat_sh, gat_out, ROWS_C, tid, cid)
    _out_acc(erep_s, CHG, den_sh, den_out, ROWS_C, tid, cid)


def _sc_gat(hs, ei_tc, als_rep, ald_rep, amax_t, amax_c):
    mesh = plsc.VectorSubcoreMesh(core_axis_name="c", subcore_axis_name="s")
    fn = pl.kernel(
        _sc_gat_body,
        out_type=[
            jax.ShapeDtypeStruct((NSC, C_SH, D), jnp.float32),
            jax.ShapeDtypeStruct((NSC, C_SH, 16), jnp.float32),
        ],
        mesh=mesh,
        compiler_params=_SC_PARAMS,
        scratch_types=[
            pltpu.VMEM((CHG,), jnp.int32),
            pltpu.VMEM((CHG,), jnp.int32),
            pltpu.VMEM((CHG,), jnp.int32),
            pltpu.VMEM((CHG, D), jnp.float32),
            pltpu.VMEM((CHG, 16), jnp.float32),
            pltpu.VMEM((CHG, 16), jnp.float32),
            pltpu.VMEM((NTILE, 16), jnp.float32),
            pltpu.SemaphoreType.DMA,
            pltpu.SemaphoreType.DMA,
            pltpu.SemaphoreType.DMA,
            pltpu.SemaphoreType.DMA,
            pltpu.SemaphoreType.DMA,
            pltpu.VMEM_SHARED((C_SH, D), jnp.float32),
            pltpu.VMEM_SHARED((C_SH, 16), jnp.float32),
        ],
    )
    return fn(hs, ei_tc, als_rep, ald_rep, amax_t, amax_c)


def kernel(x_tweet, x_cls, edge_index_tweet_to_tweet, edge_index_tweet_agg_cls,
           params):
    ei_tt = edge_index_tweet_to_tweet.astype(jnp.int32)
    ei_tc = edge_index_tweet_agg_cls.astype(jnp.int32)
    xt, xc = x_tweet, x_cls
    n = len(params)
    for i, p in enumerate(params):
        hs, als_rep, amax_t, xwr = _tc_pre_t(xt, p['gat_W'], p['gat_as'],
                                             p['sage_Wr'], p['sage_b'])
        ald_rep, amax_c = _tc_pre_c(xc, p['gat_W'], p['gat_ad'])
        agg, cnt_part = _sc_sage(xt, ei_tt)
        gat_acc, den_part = _sc_gat(hs, ei_tc, als_rep, ald_rep,
                                    amax_t, amax_c)
        act = i < n - 1
        xt = _tc_post_t(agg, cnt_part, xwr, p['sage_Wl'], act)
        xc = _tc_post_c(gat_acc, den_part, p['gat_b'], act)
    return xt, xc
